# baseline (speedup 1.0000x reference)
import functools

import jax
import jax.numpy as jnp
from jax import lax
from jax.experimental import pallas as pl
from jax.experimental.pallas import tpu as pltpu
from jax.experimental.pallas import tpu_sc as plsc

D_MODEL = 4096
N_EXPERTS = 64
TOP_K = 8
N_TOKENS = 8192
BT = 512

_SC_INFO = plsc.get_sparse_core_info()
_NC = _SC_INFO.num_cores
_NS = _SC_INFO.num_subcores
_NW = _NC * _NS
_TOK_PER_W = N_TOKENS // _NW


def _gate_body(x_ref, wt_ref, b_ref, g_ref, m_ref, z_ref, m_acc, z_acc):
    i = pl.program_id(0)

    @pl.when(i == 0)
    def _():
        m_acc[...] = jnp.full((1, N_EXPERTS), -jnp.inf, jnp.float32)
        z_acc[...] = jnp.zeros((1, N_EXPERTS), jnp.float32)

    g = jnp.dot(x_ref[...], wt_ref[...], preferred_element_type=jnp.float32)
    g = g + b_ref[...]
    g_ref[...] = g

    m_old = m_acc[...]
    m_new = jnp.maximum(m_old, jnp.max(g, axis=0, keepdims=True))
    z_acc[...] = (z_acc[...] * jnp.exp(m_old - m_new)
                  + jnp.sum(jnp.exp(g - m_new), axis=0, keepdims=True))
    m_acc[...] = m_new

    @pl.when(i == pl.num_programs(0) - 1)
    def _():
        m_ref[...] = m_acc[...]
        z_ref[...] = z_acc[...]


def _gate_logits(x, W, b):
    wt = W.T
    b2 = b.reshape(1, N_EXPERTS)
    grid = N_TOKENS // BT
    return pl.pallas_call(
        _gate_body,
        grid=(grid,),
        in_specs=[
            pl.BlockSpec((BT, D_MODEL), lambda i: (i, 0)),
            pl.BlockSpec((D_MODEL, N_EXPERTS), lambda i: (0, 0)),
            pl.BlockSpec((1, N_EXPERTS), lambda i: (0, 0)),
        ],
        out_specs=[
            pl.BlockSpec((BT, N_EXPERTS), lambda i: (i, 0)),
            pl.BlockSpec((1, N_EXPERTS), lambda i: (0, 0)),
            pl.BlockSpec((1, N_EXPERTS), lambda i: (0, 0)),
        ],
        out_shape=[
            jax.ShapeDtypeStruct((N_TOKENS, N_EXPERTS), jnp.float32),
            jax.ShapeDtypeStruct((1, N_EXPERTS), jnp.float32),
            jax.ShapeDtypeStruct((1, N_EXPERTS), jnp.float32),
        ],
        scratch_shapes=[
            pltpu.VMEM((1, N_EXPERTS), jnp.float32),
            pltpu.VMEM((1, N_EXPERTS), jnp.float32),
        ],
    )(x, wt, b2)


def _merge16(ak, av, bk, bv):
    bk2 = lax.rev(bk, (0,))
    bv2 = lax.rev(bv, (0,))
    c = ak >= bk2
    mk = jnp.where(c, ak, bk2)
    mv = jnp.where(c, av, bv2)
    return plsc.sort_key_val(mk, mv, descending=True)


def _sc_topk_body(g_hbm, m_hbm, z_hbm, out_hbm, g_v, mz_v, out_v):
    wid = lax.axis_index("s") * _NC + lax.axis_index("c")
    base = wid * _TOK_PER_W
    pltpu.sync_copy(m_hbm, mz_v.at[pl.ds(0, N_EXPERTS)])
    pltpu.sync_copy(z_hbm, mz_v.at[pl.ds(N_EXPERTS, N_EXPERTS)])
    pltpu.sync_copy(g_hbm.at[pl.ds(base * N_EXPERTS, _TOK_PER_W * N_EXPERTS)],
                    g_v)
    lane = lax.iota(jnp.int32, 16)
    ids = [lane + (16 * j) for j in range(4)]
    ms = [mz_v[pl.ds(16 * j, 16)] for j in range(4)]
    zs = [mz_v[pl.ds(N_EXPERTS + 16 * j, 16)] for j in range(4)]

    def one_token(t):
        off = t * N_EXPERTS
        srt = []
        for j in range(4):
            s = jnp.exp(g_v[pl.ds(off + 16 * j, 16)] - ms[j]) / zs[j]
            srt.append(plsc.sort_key_val(s, ids[j], descending=True))
        m01 = _merge16(srt[0][0], srt[0][1], srt[1][0], srt[1][1])
        m23 = _merge16(srt[2][0], srt[2][1], srt[3][0], srt[3][1])
        _, fv = _merge16(m01[0], m01[1], m23[0], m23[1])
        out_v[pl.ds(t * TOP_K, 16)] = fv

    def body(t2, carry):
        one_token(2 * t2)
        one_token(2 * t2 + 1)
        return carry

    lax.fori_loop(0, _TOK_PER_W // 2, body, 0)
    pltpu.sync_copy(out_v.at[pl.ds(0, _TOK_PER_W * TOP_K)],
                    out_hbm.at[pl.ds(base * TOP_K, _TOK_PER_W * TOP_K)])


_sc_topk = functools.partial(
    pl.kernel,
    out_type=jax.ShapeDtypeStruct((N_TOKENS * TOP_K,), jnp.int32),
    mesh=plsc.VectorSubcoreMesh(core_axis_name="c", subcore_axis_name="s"),
    compiler_params=pltpu.CompilerParams(needs_layout_passes=False),
    scratch_types=[
        pltpu.VMEM((_TOK_PER_W * N_EXPERTS,), jnp.float32),
        pltpu.VMEM((2 * N_EXPERTS,), jnp.float32),
        pltpu.VMEM((_TOK_PER_W * TOP_K + 16,), jnp.int32),
    ],
)(_sc_topk_body)


def kernel(x, W, b):
    g, m, z = _gate_logits(x, W, b)
    idx_flat = _sc_topk(g.reshape(-1), m.reshape(-1), z.reshape(-1))
    return idx_flat.reshape(N_TOKENS, TOP_K)


# speedup vs baseline: 1.2743x; 1.2743x over previous
import jax
import jax.numpy as jnp
from jax import lax
from jax.experimental import pallas as pl
from jax.experimental.pallas import tpu as pltpu

D_MODEL = 4096
N_EXPERTS = 64
TOP_K = 8
N_TOKENS = 8192
BT = 512
CHUNK = 1024
N_CHUNKS = N_TOKENS // CHUNK


def _topk_chunk(s):
    iota_f = lax.broadcasted_iota(jnp.int32, (CHUNK, N_EXPERTS), 1).astype(jnp.float32)
    cur = s
    cols = []
    for _ in range(TOP_K):
        mx = jnp.max(cur, axis=1, keepdims=True)
        hit = cur == mx
        idxv = jnp.where(hit, iota_f, float(N_EXPERTS))
        idx = jnp.min(idxv, axis=1, keepdims=True)
        cols.append(idx)
        cur = jnp.where(idxv == idx, -jnp.inf, cur)
    return jnp.concatenate(cols, axis=1).astype(jnp.int32)


def _gate_body(x_ref, wt_ref, b_ref, out_ref, g_acc, m_acc, z_acc):
    i = pl.program_id(0)

    @pl.when(i == 0)
    def _():
        m_acc[...] = jnp.full((1, N_EXPERTS), -jnp.inf, jnp.float32)
        z_acc[...] = jnp.zeros((1, N_EXPERTS), jnp.float32)

    g = jnp.dot(x_ref[...], wt_ref[...], preferred_element_type=jnp.float32)
    g = g + b_ref[...]
    g_acc[pl.ds(i * BT, BT), :] = g

    m_old = m_acc[...]
    m_new = jnp.maximum(m_old, jnp.max(g, axis=0, keepdims=True))
    z_acc[...] = (z_acc[...] * jnp.exp(m_old - m_new)
                  + jnp.sum(jnp.exp(g - m_new), axis=0, keepdims=True))
    m_acc[...] = m_new

    @pl.when(i == pl.num_programs(0) - 1)
    def _():
        m = m_acc[...]
        z = z_acc[...]

        def tk_body(c, carry):
            blk = g_acc[pl.ds(c * CHUNK, CHUNK), :]
            s = jnp.exp(blk - m) / z
            out_ref[pl.ds(c * CHUNK, CHUNK), :] = _topk_chunk(s)
            return carry

        lax.fori_loop(0, N_CHUNKS, tk_body, 0)


def kernel(x, W, b):
    wt = W.T
    b2 = b.reshape(1, N_EXPERTS)
    grid = N_TOKENS // BT
    return pl.pallas_call(
        _gate_body,
        grid=(grid,),
        in_specs=[
            pl.BlockSpec((BT, D_MODEL), lambda i: (i, 0)),
            pl.BlockSpec((D_MODEL, N_EXPERTS), lambda i: (0, 0)),
            pl.BlockSpec((1, N_EXPERTS), lambda i: (0, 0)),
        ],
        out_specs=pl.BlockSpec((N_TOKENS, TOP_K), lambda i: (0, 0)),
        out_shape=jax.ShapeDtypeStruct((N_TOKENS, TOP_K), jnp.int32),
        scratch_shapes=[
            pltpu.VMEM((N_TOKENS, N_EXPERTS), jnp.float32),
            pltpu.VMEM((1, N_EXPERTS), jnp.float32),
            pltpu.VMEM((1, N_EXPERTS), jnp.float32),
        ],
    )(x, wt, b2)


# speedup vs baseline: 1.2797x; 1.0042x over previous
import jax
import jax.numpy as jnp
from jax import lax
from jax.experimental import pallas as pl
from jax.experimental.pallas import tpu as pltpu

D_MODEL = 4096
N_EXPERTS = 64
TOP_K = 8
N_TOKENS = 8192
BT = 512
CHUNK = 2048
N_CHUNKS = N_TOKENS // CHUNK


def _topk_chunk(s):
    iota_f = lax.broadcasted_iota(jnp.int32, (CHUNK, N_EXPERTS), 1).astype(jnp.float32)
    cur = s
    cols = []
    for _ in range(TOP_K):
        mx = jnp.max(cur, axis=1, keepdims=True)
        hit = cur == mx
        idxv = jnp.where(hit, iota_f, float(N_EXPERTS))
        idx = jnp.min(idxv, axis=1, keepdims=True)
        cols.append(idx)
        cur = jnp.where(idxv == idx, -jnp.inf, cur)
    return jnp.concatenate(cols, axis=1).astype(jnp.int32)


def _gate_body(x_ref, wt_ref, b_ref, out_ref, g_acc, m_acc, z_acc):
    i = pl.program_id(0)

    @pl.when(i == 0)
    def _():
        m_acc[...] = jnp.full((1, N_EXPERTS), -jnp.inf, jnp.float32)
        z_acc[...] = jnp.zeros((1, N_EXPERTS), jnp.float32)

    g = jnp.dot(x_ref[...], wt_ref[...], preferred_element_type=jnp.float32)
    g = g + b_ref[...]
    g_acc[pl.ds(i * BT, BT), :] = g

    m_old = m_acc[...]
    m_new = jnp.maximum(m_old, jnp.max(g, axis=0, keepdims=True))
    z_acc[...] = (z_acc[...] * jnp.exp(m_old - m_new)
                  + jnp.sum(jnp.exp(g - m_new), axis=0, keepdims=True))
    m_acc[...] = m_new

    @pl.when(i == pl.num_programs(0) - 1)
    def _():
        m = m_acc[...]
        z = z_acc[...]

        def tk_body(c, carry):
            blk = g_acc[pl.ds(c * CHUNK, CHUNK), :]
            s = jnp.exp(blk - m) / z
            out_ref[pl.ds(c * CHUNK, CHUNK), :] = _topk_chunk(s)
            return carry

        lax.fori_loop(0, N_CHUNKS, tk_body, 0)


def kernel(x, W, b):
    wt = W.T
    b2 = b.reshape(1, N_EXPERTS)
    grid = N_TOKENS // BT
    return pl.pallas_call(
        _gate_body,
        grid=(grid,),
        in_specs=[
            pl.BlockSpec((BT, D_MODEL), lambda i: (i, 0)),
            pl.BlockSpec((D_MODEL, N_EXPERTS), lambda i: (0, 0)),
            pl.BlockSpec((1, N_EXPERTS), lambda i: (0, 0)),
        ],
        out_specs=pl.BlockSpec((N_TOKENS, TOP_K), lambda i: (0, 0)),
        out_shape=jax.ShapeDtypeStruct((N_TOKENS, TOP_K), jnp.int32),
        scratch_shapes=[
            pltpu.VMEM((N_TOKENS, N_EXPERTS), jnp.float32),
            pltpu.VMEM((1, N_EXPERTS), jnp.float32),
            pltpu.VMEM((1, N_EXPERTS), jnp.float32),
        ],
    )(x, wt, b2)
